# Initial kernel scaffold; baseline (speedup 1.0000x reference)
import jax
import jax.numpy as jnp
from jax.experimental import pallas as pl


def kernel(features, indices, weight, bias):
    raise NotImplementedError("write your pallas kernel here")


# speedup vs baseline: 31.8622x; 31.8622x over previous
import functools

import jax
import jax.numpy as jnp
from jax import lax
from jax.experimental import pallas as pl
from jax.experimental.pallas import tpu as pltpu
from jax.experimental.pallas import tpu_sc as plsc

N = 100000
CIN = 32
COUT = 32
BASE = 201
NW = 32
CHUNK = 128
K_CH = 25
N_PAD = NW * CHUNK * K_CH
HI0 = 131072
PAD_KEY = 2**31 - 65536
MAXI = 2**31 - 1


def _tc_center(feat_pad, w13, bias2d):
    blk = 2048
    assert N_PAD % blk == 0

    def body(f_ref, w_ref, b_ref, o_ref):
        o_ref[...] = (
            jnp.dot(f_ref[...], w_ref[...], preferred_element_type=jnp.float32)
            + b_ref[...]
        )

    return pl.pallas_call(
        body,
        grid=(N_PAD // blk,),
        in_specs=[
            pl.BlockSpec((blk, CIN), lambda i: (i, jnp.int32(0))),
            pl.BlockSpec((CIN, COUT), lambda i: (jnp.int32(0), jnp.int32(0))),
            pl.BlockSpec((1, COUT), lambda i: (jnp.int32(0), jnp.int32(0))),
        ],
        out_specs=pl.BlockSpec((blk, COUT), lambda i: (i, jnp.int32(0))),
        out_shape=jax.ShapeDtypeStruct((N_PAD, COUT), jnp.float32),
    )(feat_pad, w13, bias2d)


def _make_sc_kernel():
    mesh = plsc.VectorSubcoreMesh(core_axis_name="c", subcore_axis_name="s")

    @functools.partial(
        pl.kernel,
        mesh=mesh,
        out_type=jax.ShapeDtypeStruct((N_PAD, COUT), jnp.float32),
        compiler_params=pltpu.CompilerParams(needs_layout_passes=False),
        scratch_types=[
            pltpu.VMEM((N,), jnp.int32),
            pltpu.VMEM((CHUNK,), jnp.int32),
            pltpu.VMEM((CHUNK, COUT), jnp.float32),
            pltpu.VMEM((16,), jnp.int32),
            pltpu.VMEM((16,), jnp.int32),
            pltpu.VMEM((16, 128), jnp.float32),
            pltpu.VMEM((CIN, 128), jnp.float32),
            pltpu.SemaphoreType.DMA,
        ],
    )
    def sc_fn(skeys_hbm, sp_hbm, keys_hbm, feat_hbm, wf_hbm, out0_hbm, out_hbm,
              skv, kv, ov, posv, nbrv, rowsv, wkv, sem):
        wid = (lax.axis_index("s") * 2 + lax.axis_index("c")).astype(jnp.int32)
        pltpu.sync_copy(skeys_hbm, skv)
        lanes = lax.iota(jnp.int32, 16)

        def off_body(ki, _):
            @pl.when(ki != 13)
            def _():
                dx = ki // 9 - 1
                dy = (ki // 3) % 3 - 1
                dz = ki % 3 - 1
                delta = dx * (BASE * BASE) + dy * BASE + dz

                def grp_body(g, _):
                    q = kv[pl.ds(g * 16, 16)] + delta
                    lo = jnp.zeros((16,), jnp.int32)
                    hi = jnp.full((16,), HI0, jnp.int32)
                    for _s in range(18):
                        mid = (lo + hi) >> 1
                        midc = jnp.minimum(mid, N - 1)
                        v = plsc.load_gather(skv, [midc])
                        v = jnp.where(mid < N, v, MAXI)
                        upd = lo < hi
                        gt = (v < q) & upd
                        lo = jnp.where(gt, mid + 1, lo)
                        hi = jnp.where(upd & (~gt), mid, hi)
                    pc = jnp.minimum(lo, N - 1)
                    vf = plsc.load_gather(skv, [pc])
                    match = vf == q
                    mi = match.astype(jnp.int32)
                    m = jnp.sum(mi, dtype=jnp.int32)

                    @pl.when(m > 0)
                    def _():
                        posv[...] = pc
                        pltpu.async_copy(sp_hbm.at[posv], nbrv, sem).wait()
                        pltpu.async_copy(feat_hbm.at[nbrv], rowsv, sem).wait()
                        pltpu.sync_copy(wf_hbm.at[ki], wkv)

                        def lane_body(l, _):
                            ml = jnp.sum(jnp.where(lanes == l, mi, 0),
                                         dtype=jnp.int32)

                            @pl.when(ml > 0)
                            def _():
                                row = g * 16 + l
                                a0 = ov[row, pl.ds(0, 16)]
                                a1 = ov[row, pl.ds(16, 16)]

                                def cin_body(c2, carry):
                                    b0, b1 = carry
                                    lsp = jnp.full((16,), l, jnp.int32)
                                    csp = jnp.full((16,), c2, jnp.int32)
                                    fs = plsc.load_gather(rowsv, [lsp, csp])
                                    b0 = b0 + fs * wkv[c2, pl.ds(0, 16)]
                                    b1 = b1 + fs * wkv[c2, pl.ds(16, 16)]
                                    return (b0, b1)

                                a0, a1 = lax.fori_loop(
                                    jnp.int32(0), jnp.int32(CIN), cin_body,
                                    (a0, a1))
                                ov[row, pl.ds(0, 16)] = a0
                                ov[row, pl.ds(16, 16)] = a1

                            return jnp.int32(0)

                        lax.fori_loop(jnp.int32(0), jnp.int32(16), lane_body,
                                      jnp.int32(0))

                    return jnp.int32(0)

                lax.fori_loop(jnp.int32(0), jnp.int32(CHUNK // 16), grp_body,
                              jnp.int32(0))

            return jnp.int32(0)

        def chunk_body(t, _):
            base = (t * NW + wid) * CHUNK
            pltpu.sync_copy(keys_hbm.at[pl.ds(base, CHUNK)], kv)
            pltpu.sync_copy(out0_hbm.at[pl.ds(base, CHUNK)], ov)
            lax.fori_loop(jnp.int32(0), jnp.int32(27), off_body, jnp.int32(0))
            pltpu.sync_copy(ov, out_hbm.at[pl.ds(base, CHUNK)])
            return jnp.int32(0)

        lax.fori_loop(jnp.int32(0), jnp.int32(K_CH), chunk_body, jnp.int32(0))

    return sc_fn


_SC_KERNEL = _make_sc_kernel()


def kernel(features, indices, weight, bias):
    idx32 = indices.astype(jnp.int32)
    keys = ((idx32[:, 0] * BASE + idx32[:, 1]) * BASE + idx32[:, 2]) * BASE \
        + idx32[:, 3]
    sp = jnp.argsort(keys).astype(jnp.int32)
    skeys = keys[sp]
    wf = jnp.transpose(weight, (2, 3, 4, 1, 0)).reshape(27, CIN, COUT)
    wf = wf.astype(jnp.float32)
    keys_pad = jnp.concatenate(
        [keys, jnp.full((N_PAD - N,), PAD_KEY, jnp.int32)])
    feat_pad = jnp.concatenate(
        [features.astype(jnp.float32),
         jnp.zeros((N_PAD - N, CIN), jnp.float32)])
    feat_sc = jnp.zeros((N_PAD, 128), jnp.float32).at[:N, :CIN].set(
        features.astype(jnp.float32))
    wf_sc = jnp.zeros((27, CIN, 128), jnp.float32).at[:, :, :COUT].set(wf)
    out0 = _tc_center(feat_pad, wf[13],
                      bias.reshape(1, COUT).astype(jnp.float32))
    out = _SC_KERNEL(skeys, sp, keys_pad, feat_sc, wf_sc, out0)
    return out[:N]


# speedup vs baseline: 123.9128x; 3.8890x over previous
import functools

import jax
import jax.numpy as jnp
from jax import lax
from jax.experimental import pallas as pl
from jax.experimental.pallas import tpu as pltpu
from jax.experimental.pallas import tpu_sc as plsc

N = 100000
CIN = 32
COUT = 32
BASE = 201
RAD = BASE * BASE + BASE + 1
NW = 32
CHUNK = 128
K_CH = 25
N_PAD = NW * CHUNK * K_CH
HI0 = 131072
PAD_KEY = 2**31 - 65536
MAXI = 2**31 - 1
W = 16


def _tc_center(feat_pad, w13, bias2d):
    blk = 2048
    assert N_PAD % blk == 0

    def body(f_ref, w_ref, b_ref, o_ref):
        o_ref[...] = (
            jnp.dot(f_ref[...], w_ref[...], preferred_element_type=jnp.float32)
            + b_ref[...]
        )

    return pl.pallas_call(
        body,
        grid=(N_PAD // blk,),
        in_specs=[
            pl.BlockSpec((blk, CIN), lambda i: (i, jnp.int32(0))),
            pl.BlockSpec((CIN, COUT), lambda i: (jnp.int32(0), jnp.int32(0))),
            pl.BlockSpec((1, COUT), lambda i: (jnp.int32(0), jnp.int32(0))),
        ],
        out_specs=pl.BlockSpec((blk, COUT), lambda i: (i, jnp.int32(0))),
        out_shape=jax.ShapeDtypeStruct((N_PAD, COUT), jnp.float32),
    )(feat_pad, w13, bias2d)


def _decompose(kw, key):
    e = kw - key + RAD
    dxp = ((e >= 2 * (BASE * BASE)).astype(jnp.int32)
           + (e >= BASE * BASE).astype(jnp.int32))
    r2 = e - dxp * (BASE * BASE)
    dyp = ((r2 >= 2 * BASE).astype(jnp.int32)
           + (r2 >= BASE).astype(jnp.int32))
    dzp = r2 - dyp * BASE
    ki = dxp * 9 + dyp * 3 + dzp
    hit = ((e >= 0) & (r2 <= 2 * BASE + 2) & (dzp <= 2) & (ki != 13))
    return hit, ki


def _search_step(skv, q):
    def step(_i, carry):
        lo, hi = carry
        mid = (lo + hi) >> 1
        midc = jnp.minimum(mid, N - 1)
        v = plsc.load_gather(skv, [midc])
        v = jnp.where(mid < N, v, MAXI)
        upd = lo < hi
        gt = (v < q) & upd
        lo = jnp.where(gt, mid + 1, lo)
        hi = jnp.where(upd & (~gt), mid, hi)
        return (lo, hi)
    return step


def _make_sc_kernel():
    mesh = plsc.VectorSubcoreMesh(core_axis_name="c", subcore_axis_name="s")

    @functools.partial(
        pl.kernel,
        mesh=mesh,
        out_type=jax.ShapeDtypeStruct((N_PAD, COUT), jnp.float32),
        compiler_params=pltpu.CompilerParams(needs_layout_passes=False),
        scratch_types=[
            pltpu.VMEM((N,), jnp.int32),
            pltpu.VMEM((CHUNK,), jnp.int32),
            pltpu.VMEM((CHUNK, COUT), jnp.float32),
            pltpu.VMEM((16,), jnp.int32),
            pltpu.VMEM((16,), jnp.int32),
            pltpu.VMEM((16, 128), jnp.float32),
            pltpu.VMEM((CIN, 128), jnp.float32),
            pltpu.SemaphoreType.DMA,
        ],
    )
    def sc_fn(skeys_hbm, sp_hbm, keys_hbm, feat_hbm, wf_hbm, out0_hbm, out_hbm,
              skv, kv, ov, posv, nbrv, rowsv, wkv, sem):
        wid = (lax.axis_index("s") * 2 + lax.axis_index("c")).astype(jnp.int32)
        pltpu.sync_copy(skeys_hbm, skv)
        lanes = lax.iota(jnp.int32, 16)

        def accumulate(g, hitm, kiv, pos):
            hm = hitm.astype(jnp.int32)
            posv[...] = jnp.minimum(pos, N - 1)
            pltpu.async_copy(sp_hbm.at[posv], nbrv, sem).wait()
            pltpu.async_copy(feat_hbm.at[nbrv], rowsv, sem).wait()

            def lane_body(l, _):
                ml = jnp.sum(jnp.where(lanes == l, hm, 0), dtype=jnp.int32)

                @pl.when(ml > 0)
                def _():
                    kil = jnp.sum(jnp.where(lanes == l, kiv, 0),
                                  dtype=jnp.int32)
                    pltpu.sync_copy(wf_hbm.at[kil], wkv)
                    row = g * 16 + l
                    a0 = ov[row, pl.ds(0, 16)]
                    a1 = ov[row, pl.ds(16, 16)]

                    def cin_body(c2, carry):
                        b0, b1 = carry
                        lsp = jnp.full((16,), l, jnp.int32)
                        csp = jnp.full((16,), c2, jnp.int32)
                        fs = plsc.load_gather(rowsv, [lsp, csp])
                        b0 = b0 + fs * wkv[c2, pl.ds(0, 16)]
                        b1 = b1 + fs * wkv[c2, pl.ds(16, 16)]
                        return (b0, b1)

                    a0, a1 = lax.fori_loop(jnp.int32(0), jnp.int32(CIN),
                                           cin_body, (a0, a1))
                    ov[row, pl.ds(0, 16)] = a0
                    ov[row, pl.ds(16, 16)] = a1

                return jnp.int32(0)

            lax.fori_loop(jnp.int32(0), jnp.int32(16), lane_body, jnp.int32(0))

        def grp_body(g, _):
            key = kv[pl.ds(g * 16, 16)]
            qmin = key - RAD
            lo = jnp.zeros((16,), jnp.int32)
            hi = jnp.full((16,), HI0, jnp.int32)
            lo, hi = lax.fori_loop(jnp.int32(0), jnp.int32(18),
                                   _search_step(skv, qmin), (lo, hi))
            p = lo
            pe = p + (W - 1)
            vend = plsc.load_gather(skv, [jnp.minimum(pe, N - 1)])
            vend = jnp.where(pe < N, vend, MAXI)
            okv = vend >= key + RAD
            n_ok = jnp.sum(okv.astype(jnp.int32), dtype=jnp.int32)

            def scan1(w, carry):
                anyhit, prev = carry
                pw = p + w
                kw = plsc.load_gather(skv, [jnp.minimum(pw, N - 1)])
                kw = jnp.where(pw < N, kw, MAXI)
                hit, _ki = _decompose(kw, key)
                hit = hit & okv & (kw != prev)
                return (anyhit | hit.astype(jnp.int32), kw)

            prev0 = jnp.full((16,), -1, jnp.int32)
            anyhit, _ = lax.fori_loop(
                jnp.int32(0), jnp.int32(W), scan1,
                (jnp.zeros((16,), jnp.int32), prev0))
            nhit = jnp.sum(anyhit, dtype=jnp.int32)

            @pl.when(nhit > 0)
            def _():
                def scan2(w, prev):
                    pw = p + w
                    kw = plsc.load_gather(skv, [jnp.minimum(pw, N - 1)])
                    kw = jnp.where(pw < N, kw, MAXI)
                    hit, ki = _decompose(kw, key)
                    hit = hit & okv & (kw != prev)
                    mh = jnp.sum(hit.astype(jnp.int32), dtype=jnp.int32)

                    @pl.when(mh > 0)
                    def _():
                        accumulate(g, hit, ki, pw)

                    return kw

                lax.fori_loop(jnp.int32(0), jnp.int32(W), scan2, prev0)

            @pl.when(n_ok < 16)
            def _():
                def off_body(ki, _):
                    @pl.when(ki != 13)
                    def _():
                        dx = ki // 9 - 1
                        dy = (ki // 3) % 3 - 1
                        dz = ki % 3 - 1
                        delta = dx * (BASE * BASE) + dy * BASE + dz
                        q = key + delta
                        flo = jnp.zeros((16,), jnp.int32)
                        fhi = jnp.full((16,), HI0, jnp.int32)
                        flo, fhi = lax.fori_loop(
                            jnp.int32(0), jnp.int32(18),
                            _search_step(skv, q), (flo, fhi))
                        pc = jnp.minimum(flo, N - 1)
                        vf = plsc.load_gather(skv, [pc])
                        match = (vf == q) & (~okv)
                        mm = jnp.sum(match.astype(jnp.int32), dtype=jnp.int32)

                        @pl.when(mm > 0)
                        def _():
                            kvec = jnp.full((16,), ki, jnp.int32)
                            accumulate(g, match, kvec, flo)

                    return jnp.int32(0)

                lax.fori_loop(jnp.int32(0), jnp.int32(27), off_body,
                              jnp.int32(0))

            return jnp.int32(0)

        def chunk_body(t, _):
            base = (t * NW + wid) * CHUNK
            pltpu.sync_copy(keys_hbm.at[pl.ds(base, CHUNK)], kv)
            pltpu.sync_copy(out0_hbm.at[pl.ds(base, CHUNK)], ov)
            lax.fori_loop(jnp.int32(0), jnp.int32(CHUNK // 16), grp_body,
                          jnp.int32(0))
            pltpu.sync_copy(ov, out_hbm.at[pl.ds(base, CHUNK)])
            return jnp.int32(0)

        lax.fori_loop(jnp.int32(0), jnp.int32(K_CH), chunk_body, jnp.int32(0))

    return sc_fn


_SC_KERNEL = _make_sc_kernel()


def kernel(features, indices, weight, bias):
    idx32 = indices.astype(jnp.int32)
    keys = ((idx32[:, 0] * BASE + idx32[:, 1]) * BASE + idx32[:, 2]) * BASE \
        + idx32[:, 3]
    sp = jnp.argsort(keys).astype(jnp.int32)
    skeys = keys[sp]
    wf = jnp.transpose(weight, (2, 3, 4, 1, 0)).reshape(27, CIN, COUT)
    wf = wf.astype(jnp.float32)
    keys_pad = jnp.concatenate(
        [keys, jnp.full((N_PAD - N,), PAD_KEY, jnp.int32)])
    feat_pad = jnp.concatenate(
        [features.astype(jnp.float32),
         jnp.zeros((N_PAD - N, CIN), jnp.float32)])
    feat_sc = jnp.zeros((N_PAD, 128), jnp.float32).at[:N, :CIN].set(
        features.astype(jnp.float32))
    wf_sc = jnp.zeros((27, CIN, 128), jnp.float32).at[:, :, :COUT].set(wf)
    out0 = _tc_center(feat_pad, wf[13],
                      bias.reshape(1, COUT).astype(jnp.float32))
    out = _SC_KERNEL(skeys, sp, keys_pad, feat_sc, wf_sc, out0)
    return out[:N]


# speedup vs baseline: 131.9790x; 1.0651x over previous
import functools

import jax
import jax.numpy as jnp
from jax import lax
from jax.experimental import pallas as pl
from jax.experimental.pallas import tpu as pltpu
from jax.experimental.pallas import tpu_sc as plsc

N = 100000
CIN = 32
COUT = 32
BASE = 201
RAD = BASE * BASE + BASE + 1
NW = 32
CHUNK = 128
K_CH = 25
N_PAD = NW * CHUNK * K_CH
HI0 = 131072
PAD_KEY = 2**31 - 65536
MAXI = 2**31 - 1
W = 16


def _tc_center(feat_pad, w13, bias2d):
    blk = 2048
    assert N_PAD % blk == 0

    def body(f_ref, w_ref, b_ref, o_ref):
        o_ref[...] = (
            jnp.dot(f_ref[...], w_ref[...], preferred_element_type=jnp.float32)
            + b_ref[...]
        )

    return pl.pallas_call(
        body,
        grid=(N_PAD // blk,),
        in_specs=[
            pl.BlockSpec((blk, CIN), lambda i: (i, jnp.int32(0))),
            pl.BlockSpec((CIN, COUT), lambda i: (jnp.int32(0), jnp.int32(0))),
            pl.BlockSpec((1, COUT), lambda i: (jnp.int32(0), jnp.int32(0))),
        ],
        out_specs=pl.BlockSpec((blk, COUT), lambda i: (i, jnp.int32(0))),
        out_shape=jax.ShapeDtypeStruct((N_PAD, COUT), jnp.float32),
    )(feat_pad, w13, bias2d)


def _decompose(kw, key):
    e = kw - key + RAD
    dxp = ((e >= 2 * (BASE * BASE)).astype(jnp.int32)
           + (e >= BASE * BASE).astype(jnp.int32))
    r2 = e - dxp * (BASE * BASE)
    dyp = ((r2 >= 2 * BASE).astype(jnp.int32)
           + (r2 >= BASE).astype(jnp.int32))
    dzp = r2 - dyp * BASE
    ki = dxp * 9 + dyp * 3 + dzp
    hit = ((e >= 0) & (r2 <= 2 * BASE + 2) & (dzp <= 2) & (ki != 13))
    return hit, ki


def _search_step(skv, q):
    def step(_i, carry):
        lo, hi = carry
        mid = (lo + hi) >> 1
        midc = jnp.minimum(mid, N - 1)
        v = plsc.load_gather(skv, [midc])
        v = jnp.where(mid < N, v, MAXI)
        upd = lo < hi
        gt = (v < q) & upd
        lo = jnp.where(gt, mid + 1, lo)
        hi = jnp.where(upd & (~gt), mid, hi)
        return (lo, hi)
    return step


def _make_sc_kernel():
    mesh = plsc.VectorSubcoreMesh(core_axis_name="c", subcore_axis_name="s")

    @functools.partial(
        pl.kernel,
        mesh=mesh,
        out_type=jax.ShapeDtypeStruct((N_PAD, COUT), jnp.float32),
        compiler_params=pltpu.CompilerParams(needs_layout_passes=False),
        scratch_types=[
            pltpu.VMEM((N,), jnp.int32),
            pltpu.VMEM((CHUNK,), jnp.int32),
            pltpu.VMEM((CHUNK,), jnp.int32),
            pltpu.VMEM((CHUNK, COUT), jnp.float32),
            pltpu.VMEM((16,), jnp.int32),
            pltpu.VMEM((16,), jnp.int32),
            pltpu.VMEM((16, 128), jnp.float32),
            pltpu.VMEM((CIN, 128), jnp.float32),
            pltpu.SemaphoreType.DMA,
        ],
    )
    def sc_fn(skeys_hbm, sp_hbm, keys_hbm, feat_hbm, wf_hbm, out0_hbm, out_hbm,
              skv, kv, pv, ov, posv, nbrv, rowsv, wkv, sem):
        wid = (lax.axis_index("s") * 2 + lax.axis_index("c")).astype(jnp.int32)
        pltpu.sync_copy(skeys_hbm, skv)
        lanes = lax.iota(jnp.int32, 16)

        def accumulate(g, hitm, kiv, pos):
            hm = hitm.astype(jnp.int32)
            posv[...] = jnp.minimum(pos, N - 1)
            pltpu.async_copy(sp_hbm.at[posv], nbrv, sem).wait()
            pltpu.async_copy(feat_hbm.at[nbrv], rowsv, sem).wait()

            def lane_body(l, _):
                ml = jnp.sum(jnp.where(lanes == l, hm, 0), dtype=jnp.int32)

                @pl.when(ml > 0)
                def _():
                    kil = jnp.sum(jnp.where(lanes == l, kiv, 0),
                                  dtype=jnp.int32)
                    pltpu.sync_copy(wf_hbm.at[kil], wkv)
                    row = g * 16 + l
                    a0 = ov[row, pl.ds(0, 16)]
                    a1 = ov[row, pl.ds(16, 16)]

                    def cin_body(c2, carry):
                        b0, b1 = carry
                        lsp = jnp.full((16,), l, jnp.int32)
                        csp = jnp.full((16,), c2, jnp.int32)
                        fs = plsc.load_gather(rowsv, [lsp, csp])
                        b0 = b0 + fs * wkv[c2, pl.ds(0, 16)]
                        b1 = b1 + fs * wkv[c2, pl.ds(16, 16)]
                        return (b0, b1)

                    a0, a1 = lax.fori_loop(jnp.int32(0), jnp.int32(CIN),
                                           cin_body, (a0, a1))
                    ov[row, pl.ds(0, 16)] = a0
                    ov[row, pl.ds(16, 16)] = a1

                return jnp.int32(0)

            lax.fori_loop(jnp.int32(0), jnp.int32(16), lane_body, jnp.int32(0))

        def grp_body(g, _):
            key = kv[pl.ds(g * 16, 16)]
            p = pv[pl.ds(g * 16, 16)]
            pe = p + (W - 1)
            vend = plsc.load_gather(skv, [jnp.minimum(pe, N - 1)])
            vend = jnp.where(pe < N, vend, MAXI)
            okv = vend >= key + RAD

            def scan1(w, carry):
                anyhit, prev = carry
                pw = p + w
                kw = plsc.load_gather(skv, [jnp.minimum(pw, N - 1)])
                kw = jnp.where(pw < N, kw, MAXI)
                hit, _ki = _decompose(kw, key)
                hit = hit & okv & (kw != prev)
                return (anyhit | hit.astype(jnp.int32), kw)

            prev0 = jnp.full((16,), -1, jnp.int32)
            anyhit, _ = lax.fori_loop(
                jnp.int32(0), jnp.int32(W), scan1,
                (jnp.zeros((16,), jnp.int32), prev0))
            notok = jnp.int32(1) - okv.astype(jnp.int32)
            comb = jnp.sum(anyhit + (notok << 8), dtype=jnp.int32)
            nhit = comb & 255
            nbad = comb >> 8

            @pl.when(nhit > 0)
            def _():
                def scan2(w, prev):
                    pw = p + w
                    kw = plsc.load_gather(skv, [jnp.minimum(pw, N - 1)])
                    kw = jnp.where(pw < N, kw, MAXI)
                    hit, ki = _decompose(kw, key)
                    hit = hit & okv & (kw != prev)
                    mh = jnp.sum(hit.astype(jnp.int32), dtype=jnp.int32)

                    @pl.when(mh > 0)
                    def _():
                        accumulate(g, hit, ki, pw)

                    return kw

                lax.fori_loop(jnp.int32(0), jnp.int32(W), scan2, prev0)

            @pl.when(nbad > 0)
            def _():
                def off_body(ki, _):
                    @pl.when(ki != 13)
                    def _():
                        dx = ki // 9 - 1
                        dy = (ki // 3) % 3 - 1
                        dz = ki % 3 - 1
                        delta = dx * (BASE * BASE) + dy * BASE + dz
                        q = key + delta
                        flo = jnp.zeros((16,), jnp.int32)
                        fhi = jnp.full((16,), HI0, jnp.int32)
                        flo, fhi = lax.fori_loop(
                            jnp.int32(0), jnp.int32(18),
                            _search_step(skv, q), (flo, fhi))
                        pc = jnp.minimum(flo, N - 1)
                        vf = plsc.load_gather(skv, [pc])
                        match = (vf == q) & (~okv)
                        mm = jnp.sum(match.astype(jnp.int32), dtype=jnp.int32)

                        @pl.when(mm > 0)
                        def _():
                            kvec = jnp.full((16,), ki, jnp.int32)
                            accumulate(g, match, kvec, flo)

                    return jnp.int32(0)

                lax.fori_loop(jnp.int32(0), jnp.int32(27), off_body,
                              jnp.int32(0))

            return jnp.int32(0)

        def chunk_body(t, _):
            base = (t * NW + wid) * CHUNK
            pltpu.sync_copy(keys_hbm.at[pl.ds(base, CHUNK)], kv)
            pltpu.sync_copy(out0_hbm.at[pl.ds(base, CHUNK)], ov)
            ng = CHUNK // 16
            los = []
            his = []
            qms = []
            for g2 in range(ng):
                qms.append(kv[pl.ds(g2 * 16, 16)] - RAD)
                los.append(jnp.zeros((16,), jnp.int32))
                his.append(jnp.full((16,), HI0, jnp.int32))
            for _s in range(18):
                for g2 in range(ng):
                    lo, hi, q = los[g2], his[g2], qms[g2]
                    mid = (lo + hi) >> 1
                    v = plsc.load_gather(skv, [jnp.minimum(mid, N - 1)])
                    v = jnp.where(mid < N, v, MAXI)
                    upd = lo < hi
                    gt = (v < q) & upd
                    los[g2] = jnp.where(gt, mid + 1, lo)
                    his[g2] = jnp.where(upd & (~gt), mid, hi)
            for g2 in range(ng):
                pv[pl.ds(g2 * 16, 16)] = los[g2]
            lax.fori_loop(jnp.int32(0), jnp.int32(CHUNK // 16), grp_body,
                          jnp.int32(0))
            pltpu.sync_copy(ov, out_hbm.at[pl.ds(base, CHUNK)])
            return jnp.int32(0)

        lax.fori_loop(jnp.int32(0), jnp.int32(K_CH), chunk_body, jnp.int32(0))

    return sc_fn


_SC_KERNEL = _make_sc_kernel()


def kernel(features, indices, weight, bias):
    idx32 = indices.astype(jnp.int32)
    keys = ((idx32[:, 0] * BASE + idx32[:, 1]) * BASE + idx32[:, 2]) * BASE \
        + idx32[:, 3]
    iota = lax.iota(jnp.int32, N)
    skeys, sp = lax.sort([keys, iota], num_keys=1, is_stable=True)
    wf = jnp.transpose(weight, (2, 3, 4, 1, 0)).reshape(27, CIN, COUT)
    wf = wf.astype(jnp.float32)
    keys_pad = jnp.concatenate(
        [keys, jnp.full((N_PAD - N,), PAD_KEY, jnp.int32)])
    feat_pad = jnp.concatenate(
        [features.astype(jnp.float32),
         jnp.zeros((N_PAD - N, CIN), jnp.float32)])
    feat_sc = jnp.zeros((N_PAD, 128), jnp.float32).at[:N, :CIN].set(
        features.astype(jnp.float32))
    wf_sc = jnp.zeros((27, CIN, 128), jnp.float32).at[:, :, :COUT].set(wf)
    out0 = _tc_center(feat_pad, wf[13],
                      bias.reshape(1, COUT).astype(jnp.float32))
    out = _SC_KERNEL(skeys, sp, keys_pad, feat_sc, wf_sc, out0)
    return out[:N]
